# baseline (speedup 1.0000x reference)
import functools

import jax
import jax.numpy as jnp
from jax.experimental import pallas as pl
from jax.experimental.pallas import tpu as pltpu


def _enc_body(nd, x_ref, pb_ref, w_ref, lb_ref, zpre_ref):
    d = pl.program_id(2)

    @pl.when(d == 0)
    def _init():
        zpre_ref[...] = jnp.zeros_like(zpre_ref)

    xs = x_ref[...] - pb_ref[...]
    zpre_ref[...] += jnp.dot(xs, w_ref[...], preferred_element_type=jnp.float32)

    @pl.when(d == nd - 1)
    def _finish():
        zpre_ref[...] += lb_ref[...]


def _topk_body(K, zpre_ref, z_ref):
    zpre = zpre_ref[...]
    ri = jax.lax.bitcast_convert_type(jnp.maximum(zpre, 0.0), jnp.int32)
    bb = ri.shape[0]
    h16 = (ri >> 16).astype(jnp.int16)

    lo0 = jnp.full((bb, 1), -1, jnp.int32)
    hi0 = jnp.full((bb, 1), 0x7F80, jnp.int32)

    def step1(_, carry):
        lo, hi = carry
        mid = lo + ((hi - lo) >> 1)
        c = jnp.sum((h16 > mid.astype(jnp.int16)).astype(jnp.float32),
                    axis=1, keepdims=True)
        big = c >= K
        return jnp.where(big, mid, lo), jnp.where(big, hi, mid)

    _, tstar = jax.lax.fori_loop(0, 15, step1, (lo0, hi0))

    t16 = tstar.astype(jnp.int16)
    u16 = (ri & 0xFFFF).astype(jnp.int16) ^ jnp.int16(-32768)
    res = jnp.where(h16 > t16, jnp.int16(32767),
                    jnp.where(h16 < t16, jnp.int16(-32768), u16))

    vlo0 = jnp.full((bb, 1), -1, jnp.int32)
    vhi0 = jnp.full((bb, 1), 65535, jnp.int32)

    def step2(_, carry):
        lo, hi = carry
        mid = lo + ((hi - lo) >> 1)
        c = jnp.sum((res > (mid - 32768).astype(jnp.int16)).astype(jnp.float32),
                    axis=1, keepdims=True)
        big = c >= K
        return jnp.where(big, mid, lo), jnp.where(big, hi, mid)

    vlo, _ = jax.lax.fori_loop(0, 16, step2, (vlo0, vhi0))

    lo32 = jnp.maximum((tstar << 16) + vlo, 0)
    z_ref[...] = jnp.where(ri > lo32, zpre, 0.0)


def _dec_body(nl, z_ref, w_ref, pb_ref, xhat_ref):
    l = pl.program_id(1)

    @pl.when(l == 0)
    def _init():
        xhat_ref[...] = jnp.zeros_like(xhat_ref)

    xhat_ref[...] += jnp.dot(z_ref[...], w_ref[...], preferred_element_type=jnp.float32)

    @pl.when(l == nl - 1)
    def _finish():
        xhat_ref[...] += pb_ref[...]


@jax.jit
def kernel(x_BD, pb_D, W_enc, lb_L, W_dec):
    B, D = x_BD.shape
    L = W_enc.shape[1]
    K = 64

    bb = min(1024, B)
    bl = min(2048, L)
    bd = min(512, D)
    nd = D // bd
    pb2 = pb_D.reshape(1, D)
    lb2 = lb_L.reshape(1, L)

    zpre_BL = pl.pallas_call(
        functools.partial(_enc_body, nd),
        grid=(B // bb, L // bl, nd),
        in_specs=[
            pl.BlockSpec((bb, bd), lambda b, l, d: (b, d)),
            pl.BlockSpec((1, bd), lambda b, l, d: (0, d)),
            pl.BlockSpec((bd, bl), lambda b, l, d: (d, l)),
            pl.BlockSpec((1, bl), lambda b, l, d: (0, l)),
        ],
        out_specs=pl.BlockSpec((bb, bl), lambda b, l, d: (b, l)),
        out_shape=jax.ShapeDtypeStruct((B, L), jnp.float32),
        compiler_params=pltpu.CompilerParams(
            dimension_semantics=("parallel", "parallel", "arbitrary"),
        ),
    )(x_BD, pb2, W_enc, lb2)

    bbm = min(256, B)
    z_BL = pl.pallas_call(
        functools.partial(_topk_body, K),
        grid=(B // bbm,),
        in_specs=[pl.BlockSpec((bbm, L), lambda b: (b, 0))],
        out_specs=pl.BlockSpec((bbm, L), lambda b: (b, 0)),
        out_shape=jax.ShapeDtypeStruct((B, L), jnp.float32),
        compiler_params=pltpu.CompilerParams(
            dimension_semantics=("parallel",),
        ),
    )(zpre_BL)

    bb2 = min(1024, B)
    bl2 = min(512, L)
    nl = L // bl2
    xhat_BD = pl.pallas_call(
        functools.partial(_dec_body, nl),
        grid=(B // bb2, nl),
        in_specs=[
            pl.BlockSpec((bb2, bl2), lambda b, l: (b, l)),
            pl.BlockSpec((bl2, D), lambda b, l: (l, 0)),
            pl.BlockSpec((1, D), lambda b, l: (0, 0)),
        ],
        out_specs=pl.BlockSpec((bb2, D), lambda b, l: (b, 0)),
        out_shape=jax.ShapeDtypeStruct((B, D), jnp.float32),
        compiler_params=pltpu.CompilerParams(
            dimension_semantics=("parallel", "arbitrary"),
        ),
    )(z_BL, W_dec, pb2)

    return (zpre_BL, z_BL, xhat_BD)


# speedup vs baseline: 1.4566x; 1.4566x over previous
import functools

import jax
import jax.numpy as jnp
from jax.experimental import pallas as pl
from jax.experimental.pallas import tpu as pltpu


def _enc_body(nd, x_ref, pb_ref, w_ref, lb_ref, zpre_ref):
    d = pl.program_id(2)

    @pl.when(d == 0)
    def _init():
        zpre_ref[...] = jnp.zeros_like(zpre_ref)

    xs = x_ref[...] - pb_ref[...]
    zpre_ref[...] += jnp.dot(xs, w_ref[...], preferred_element_type=jnp.float32)

    @pl.when(d == nd - 1)
    def _finish():
        zpre_ref[...] += lb_ref[...]


def _topk_body(K, zpre_ref, z_ref):
    zpre = zpre_ref[...]
    ri = jax.lax.bitcast_convert_type(jnp.maximum(zpre, 0.0), jnp.int32)
    bb = ri.shape[0]
    lo0 = jnp.zeros((bb, 1), jnp.int32)
    hi0 = jnp.full((bb, 1), 0x7F800000, jnp.int32)

    def step(_, carry):
        lo, hi = carry
        mid = lo + ((hi - lo) >> 1)
        c = jnp.sum((ri > mid).astype(jnp.int32), axis=1, keepdims=True)
        big = c >= K
        return jnp.where(big, mid, lo), jnp.where(big, hi, mid)

    lo, _ = jax.lax.fori_loop(0, 31, step, (lo0, hi0))
    z_ref[...] = jnp.where(ri > lo, zpre, 0.0)


def _dec_body(nl, z_ref, w_ref, pb_ref, xhat_ref):
    l = pl.program_id(1)

    @pl.when(l == 0)
    def _init():
        xhat_ref[...] = jnp.zeros_like(xhat_ref)

    xhat_ref[...] += jnp.dot(z_ref[...], w_ref[...], preferred_element_type=jnp.float32)

    @pl.when(l == nl - 1)
    def _finish():
        xhat_ref[...] += pb_ref[...]


@jax.jit
def kernel(x_BD, pb_D, W_enc, lb_L, W_dec):
    B, D = x_BD.shape
    L = W_enc.shape[1]
    K = 64

    bb = min(2048, B)
    bl = min(2048, L)
    bd = min(512, D)
    nd = D // bd
    pb2 = pb_D.reshape(1, D)
    lb2 = lb_L.reshape(1, L)

    zpre_BL = pl.pallas_call(
        functools.partial(_enc_body, nd),
        grid=(B // bb, L // bl, nd),
        in_specs=[
            pl.BlockSpec((bb, bd), lambda b, l, d: (b, d)),
            pl.BlockSpec((1, bd), lambda b, l, d: (0, d)),
            pl.BlockSpec((bd, bl), lambda b, l, d: (d, l)),
            pl.BlockSpec((1, bl), lambda b, l, d: (0, l)),
        ],
        out_specs=pl.BlockSpec((bb, bl), lambda b, l, d: (b, l)),
        out_shape=jax.ShapeDtypeStruct((B, L), jnp.float32),
        compiler_params=pltpu.CompilerParams(
            dimension_semantics=("parallel", "parallel", "arbitrary"),
        ),
    )(x_BD, pb2, W_enc, lb2)

    bbm = min(256, B)
    z_BL = pl.pallas_call(
        functools.partial(_topk_body, K),
        grid=(B // bbm,),
        in_specs=[pl.BlockSpec((bbm, L), lambda b: (b, 0))],
        out_specs=pl.BlockSpec((bbm, L), lambda b: (b, 0)),
        out_shape=jax.ShapeDtypeStruct((B, L), jnp.float32),
        compiler_params=pltpu.CompilerParams(
            dimension_semantics=("parallel",),
        ),
    )(zpre_BL)

    bb2 = min(2048, B)
    bl2 = min(512, L)
    nl = L // bl2
    xhat_BD = pl.pallas_call(
        functools.partial(_dec_body, nl),
        grid=(B // bb2, nl),
        in_specs=[
            pl.BlockSpec((bb2, bl2), lambda b, l: (b, l)),
            pl.BlockSpec((bl2, D), lambda b, l: (l, 0)),
            pl.BlockSpec((1, D), lambda b, l: (0, 0)),
        ],
        out_specs=pl.BlockSpec((bb2, D), lambda b, l: (b, 0)),
        out_shape=jax.ShapeDtypeStruct((B, D), jnp.float32),
        compiler_params=pltpu.CompilerParams(
            dimension_semantics=("parallel", "arbitrary"),
        ),
    )(z_BL, W_dec, pb2)

    return (zpre_BL, z_BL, xhat_BD)


# speedup vs baseline: 1.6777x; 1.1518x over previous
import functools

import jax
import jax.numpy as jnp
from jax.experimental import pallas as pl
from jax.experimental.pallas import tpu as pltpu


def _enc_body(nd, x_ref, pb_ref, w_ref, lb_ref, zpre_ref):
    d = pl.program_id(2)

    @pl.when(d == 0)
    def _init():
        zpre_ref[...] = jnp.zeros_like(zpre_ref)

    xs = x_ref[...] - pb_ref[...]
    zpre_ref[...] += jnp.dot(xs, w_ref[...], preferred_element_type=jnp.float32)

    @pl.when(d == nd - 1)
    def _finish():
        zpre_ref[...] += lb_ref[...]


def _topk_body(K, zpre_ref, z_ref):
    zpre = zpre_ref[...]
    bb = zpre.shape[0]
    lo0 = jnp.zeros((bb, 1), jnp.int32)
    hi0 = jnp.full((bb, 1), 0x7F800000, jnp.int32)

    def step(_, carry):
        lo, hi = carry
        mid = lo + ((hi - lo) >> 1)
        mid_f = jax.lax.bitcast_convert_type(mid, jnp.float32)
        c = jnp.sum((zpre > mid_f).astype(jnp.int32), axis=1, keepdims=True)
        big = c >= K
        return jnp.where(big, mid, lo), jnp.where(big, hi, mid)

    lo, _ = jax.lax.fori_loop(0, 31, step, (lo0, hi0))
    lo_f = jax.lax.bitcast_convert_type(lo, jnp.float32)
    z_ref[...] = jnp.where(zpre > lo_f, zpre, 0.0)


def _dec_body(nl, z_ref, w_ref, pb_ref, xhat_ref):
    l = pl.program_id(1)

    @pl.when(l == 0)
    def _init():
        xhat_ref[...] = jnp.zeros_like(xhat_ref)

    xhat_ref[...] += jnp.dot(z_ref[...], w_ref[...], preferred_element_type=jnp.float32)

    @pl.when(l == nl - 1)
    def _finish():
        xhat_ref[...] += pb_ref[...]


@jax.jit
def kernel(x_BD, pb_D, W_enc, lb_L, W_dec):
    B, D = x_BD.shape
    L = W_enc.shape[1]
    K = 64

    bb = min(2048, B)
    bl = min(2048, L)
    bd = min(512, D)
    nd = D // bd
    pb2 = pb_D.reshape(1, D)
    lb2 = lb_L.reshape(1, L)

    zpre_BL = pl.pallas_call(
        functools.partial(_enc_body, nd),
        grid=(B // bb, L // bl, nd),
        in_specs=[
            pl.BlockSpec((bb, bd), lambda b, l, d: (b, d)),
            pl.BlockSpec((1, bd), lambda b, l, d: (0, d)),
            pl.BlockSpec((bd, bl), lambda b, l, d: (d, l)),
            pl.BlockSpec((1, bl), lambda b, l, d: (0, l)),
        ],
        out_specs=pl.BlockSpec((bb, bl), lambda b, l, d: (b, l)),
        out_shape=jax.ShapeDtypeStruct((B, L), jnp.float32),
        compiler_params=pltpu.CompilerParams(
            dimension_semantics=("parallel", "parallel", "arbitrary"),
        ),
    )(x_BD, pb2, W_enc, lb2)

    bbm = min(256, B)
    z_BL = pl.pallas_call(
        functools.partial(_topk_body, K),
        grid=(B // bbm,),
        in_specs=[pl.BlockSpec((bbm, L), lambda b: (b, 0))],
        out_specs=pl.BlockSpec((bbm, L), lambda b: (b, 0)),
        out_shape=jax.ShapeDtypeStruct((B, L), jnp.float32),
        compiler_params=pltpu.CompilerParams(
            dimension_semantics=("parallel",),
        ),
    )(zpre_BL)

    bb2 = min(2048, B)
    bl2 = min(512, L)
    nl = L // bl2
    xhat_BD = pl.pallas_call(
        functools.partial(_dec_body, nl),
        grid=(B // bb2, nl),
        in_specs=[
            pl.BlockSpec((bb2, bl2), lambda b, l: (b, l)),
            pl.BlockSpec((bl2, D), lambda b, l: (l, 0)),
            pl.BlockSpec((1, D), lambda b, l: (0, 0)),
        ],
        out_specs=pl.BlockSpec((bb2, D), lambda b, l: (b, 0)),
        out_shape=jax.ShapeDtypeStruct((B, D), jnp.float32),
        compiler_params=pltpu.CompilerParams(
            dimension_semantics=("parallel", "arbitrary"),
        ),
    )(z_BL, W_dec, pb2)

    return (zpre_BL, z_BL, xhat_BD)
